# Initial kernel scaffold; baseline (speedup 1.0000x reference)
import jax
import jax.numpy as jnp
from jax.experimental import pallas as pl


def kernel(edge_index, x, cond, n_index, W1, b1, ln1_g, ln1_b, f1_W, f1_b, W2, b2, ln2_g, ln2_b, f2_W, f2_b):
    raise NotImplementedError("write your pallas kernel here")


# speedup vs baseline: 5.2106x; 5.2106x over previous
import functools

import jax
import jax.numpy as jnp
from jax import lax
from jax.experimental import pallas as pl
from jax.experimental.pallas import tpu as pltpu
from jax.experimental.pallas import tpu_sc as plsc

N = 10000
E = 160000
D = 256
H = 128
CD = 128
B = 64

NS = 16
EPT = E // NS
C3 = 80
NCH3 = EPT // C3
EPW = 2 * E // (2 * NS)
C1 = 80
NCH1 = EPW // C1
RWB = 1000

def _deg_body(edge32, zeros1, ones_h, out_hbm, table, idx2d, ones_v, wb):
    cid = lax.axis_index("c")
    sid = lax.axis_index("s")
    w = cid * NS + sid

    @pl.when(sid == 0)
    def _():
        pltpu.sync_copy(zeros1, table)

    pltpu.sync_copy(ones_h, ones_v)
    pltpu.sync_copy(edge32.at[w], idx2d)
    plsc.subcore_barrier()

    @pl.loop(0, NCH1)
    def _(j):
        pltpu.sync_copy(ones_v, table.at[idx2d.at[j]], add=True)

    plsc.subcore_barrier()

    @pl.when(sid < N // RWB)
    def _():
        r0 = sid * RWB
        pltpu.sync_copy(table.at[pl.ds(r0, RWB)], wb)
        pltpu.sync_copy(wb, out_hbm.at[pl.ds(cid * N + r0, RWB)])


def _prescale_body(x_ref, deg_ref, out_ref):
    norm = lax.rsqrt(jnp.maximum(deg_ref[...], 1.0))
    h = x_ref[...] * norm
    out_ref[0] = h[:, :H]
    out_ref[1] = h[:, H:]


def _prescale(x, deg_out_tbl):
    R = 1000
    return pl.pallas_call(
        _prescale_body,
        grid=(N // R,),
        in_specs=[
            pl.BlockSpec((R, D), lambda i: (i, 0)),
            pl.BlockSpec((R, 1), lambda i: (i, 0)),
        ],
        out_specs=pl.BlockSpec((2, R, H), lambda i: (0, i, 0)),
        out_shape=jax.ShapeDtypeStruct((2, N, H), jnp.float32),
    )(x, deg_out_tbl)


def _scat_body(src2d, dst3d, h2d, zeros_h, out_hbm, aggS, sidx, didx, rows):
    cid = lax.axis_index("c")
    sid = lax.axis_index("s")

    @pl.when(sid == 0)
    def _():
        pltpu.sync_copy(zeros_h, aggS)

    pltpu.sync_copy(src2d.at[sid], sidx)
    pltpu.sync_copy(dst3d.at[sid], didx)
    off = cid * N

    @pl.loop(0, EPT // 16)
    def _(k):
        s = pl.ds(k * 16, 16)
        sidx[s] = sidx[s] + off

    plsc.subcore_barrier()

    @pl.loop(0, NCH3)
    def _(j):
        pltpu.sync_copy(h2d.at[sidx.at[pl.ds(j * C3, C3)]], rows)
        pltpu.sync_copy(rows, aggS.at[didx.at[j]], add=True)

    plsc.subcore_barrier()

    @pl.when(sid < N // RWB)
    def _():
        r0 = sid * RWB
        pltpu.sync_copy(aggS.at[pl.ds(r0, RWB)], out_hbm.at[pl.ds(cid * N + r0, RWB)])


def _final_body(a0, a1, degin, xr, nidx, cond, f2W, f2b, W2a, W2b, b2, g2, bt2, out):
    ss = jnp.dot(cond[...], f2W[...], preferred_element_type=jnp.float32) + f2b[...]
    gamma = ss[:, :D]
    beta = ss[:, D:]
    idx = nidx[0, 0, :]
    oh = (idx[:, None] == lax.broadcasted_iota(jnp.int32, (1, B), 1)).astype(jnp.float32)
    gg = jnp.dot(oh, gamma, preferred_element_type=jnp.float32)
    bb = jnp.dot(oh, beta, preferred_element_type=jnp.float32)
    norm = lax.rsqrt(jnp.maximum(degin[...], 1.0))
    h2 = (
        jnp.dot(a0[...], W2a[...], preferred_element_type=jnp.float32)
        + jnp.dot(a1[...], W2b[...], preferred_element_type=jnp.float32)
    ) * norm + b2[...]
    mu = jnp.mean(h2, axis=1, keepdims=True)
    xc = h2 - mu
    var = jnp.mean(xc * xc, axis=1, keepdims=True)
    ln = xc * lax.rsqrt(var + 1e-5) * g2[...] + bt2[...]
    f = ln * (1.0 + gg) + bb
    out[...] = f * jax.nn.sigmoid(f) + xr[...]


def _final(agg0, agg1, deg_in_tbl, x, n_index, cond, f2_W, f2_b, W2, b2, ln2_g, ln2_b):
    R = 1000
    rep = lambda i: (0, 0)
    return pl.pallas_call(
        _final_body,
        grid=(N // R,),
        in_specs=[
            pl.BlockSpec((R, H), lambda i: (i, 0)),
            pl.BlockSpec((R, H), lambda i: (i, 0)),
            pl.BlockSpec((R, 1), lambda i: (i, 0)),
            pl.BlockSpec((R, D), lambda i: (i, 0)),
            pl.BlockSpec((1, 1, R), lambda i: (i, 0, 0)),
            pl.BlockSpec((B, CD), rep),
            pl.BlockSpec((CD, 2 * D), rep),
            pl.BlockSpec((1, 2 * D), rep),
            pl.BlockSpec((H, D), rep),
            pl.BlockSpec((H, D), rep),
            pl.BlockSpec((1, D), rep),
            pl.BlockSpec((1, D), rep),
            pl.BlockSpec((1, D), rep),
        ],
        out_specs=pl.BlockSpec((R, D), lambda i: (i, 0)),
        out_shape=jax.ShapeDtypeStruct((N, D), jnp.float32),
    )(
        agg0, agg1, deg_in_tbl, x,
        n_index.reshape(N // R, 1, R),
        cond, f2_W, f2_b.reshape(1, 2 * D),
        W2[:H], W2[H:],
        b2.reshape(1, D), ln2_g.reshape(1, D), ln2_b.reshape(1, D),
    )


@functools.lru_cache(maxsize=None)
def _sc_calls():
    mesh = plsc.VectorSubcoreMesh(
        core_axis_name="c", subcore_axis_name="s", num_cores=2, num_subcores=NS
    )
    deg_call = pl.kernel(
        _deg_body,
        out_type=jax.ShapeDtypeStruct((2 * N,), jnp.float32),
        mesh=mesh,
        scratch_types=[
            pltpu.VMEM_SHARED((N,), jnp.float32),
            pltpu.VMEM((NCH1, C1), jnp.int32),
            pltpu.VMEM((C1,), jnp.float32),
            pltpu.VMEM((RWB,), jnp.float32),
        ],
    )
    scat_call = pl.kernel(
        _scat_body,
        out_type=jax.ShapeDtypeStruct((2 * N, H), jnp.float32),
        mesh=mesh,
        scratch_types=[
            pltpu.VMEM_SHARED((N, H), jnp.float32),
            pltpu.VMEM((EPT,), jnp.int32),
            pltpu.VMEM((NCH3, C3), jnp.int32),
            pltpu.VMEM((C3, H), jnp.float32),
        ],
    )
    return deg_call, scat_call


def kernel(edge_index, x, cond, n_index, W1, b1, ln1_g, ln1_b, f1_W, f1_b,
           W2, b2, ln2_g, ln2_b, f2_W, f2_b):
    _deg_call, _scat_call = _sc_calls()
    edge32 = edge_index.reshape(2 * NS, NCH1, C1)
    degtbl = _deg_call(
        edge32,
        jnp.zeros((N,), jnp.float32),
        jnp.ones((C1,), jnp.float32),
    )
    deg2d = degtbl.reshape(2 * N, 1)

    h2d = _prescale(x, deg2d[:N]).reshape(2 * N, H)

    src2d = edge_index[0].reshape(NS, EPT)
    dst3d = edge_index[1].reshape(NS, NCH3, C3)
    agg = _scat_call(src2d, dst3d, h2d, jnp.zeros((N, H), jnp.float32))

    return _final(agg[:N], agg[N:], deg2d[N:], x, n_index, cond,
                  f2_W, f2_b, W2, b2, ln2_g, ln2_b)


# speedup vs baseline: 7.4212x; 1.4242x over previous
import functools

import jax
import jax.numpy as jnp
from jax import lax
from jax.experimental import pallas as pl
from jax.experimental.pallas import tpu as pltpu
from jax.experimental.pallas import tpu_sc as plsc

N = 10000
E = 160000
D = 256
H = 128
CD = 128
B = 64

NS = 16
EPT = E // NS
C3 = 80
NCH3 = EPT // C3
EPW = 2 * E // (2 * NS)
C1 = 80
NCH1 = EPW // C1
RWB = 1000

def _deg_body(edge32, zeros1, ones_h, out_hbm, table, idx2d, ones_v, wb):
    cid = lax.axis_index("c")
    sid = lax.axis_index("s")
    w = cid * NS + sid

    @pl.when(sid == 0)
    def _():
        pltpu.sync_copy(zeros1, table)

    pltpu.sync_copy(ones_h, ones_v)
    pltpu.sync_copy(edge32.at[w], idx2d)
    plsc.subcore_barrier()

    @pl.loop(0, NCH1)
    def _(j):
        pltpu.sync_copy(ones_v, table.at[idx2d.at[j]], add=True)

    plsc.subcore_barrier()

    @pl.when(sid < N // RWB)
    def _():
        r0 = sid * RWB
        pltpu.sync_copy(table.at[pl.ds(r0, RWB)], wb)
        pltpu.sync_copy(wb, out_hbm.at[pl.ds(cid * N + r0, RWB)])


def _prescale_body(x_ref, deg_ref, out_ref):
    norm = lax.rsqrt(jnp.maximum(deg_ref[...], 1.0))
    h = x_ref[...] * norm
    out_ref[0] = h[:, :H]
    out_ref[1] = h[:, H:]


def _prescale(x, deg_out_tbl):
    R = 1000
    return pl.pallas_call(
        _prescale_body,
        grid=(N // R,),
        in_specs=[
            pl.BlockSpec((R, D), lambda i: (i, 0)),
            pl.BlockSpec((R, 1), lambda i: (i, 0)),
        ],
        out_specs=pl.BlockSpec((2, R, H), lambda i: (0, i, 0)),
        out_shape=jax.ShapeDtypeStruct((2, N, H), jnp.float32),
    )(x, deg_out_tbl)


def _scat_body(src2d, dst3d, h2d, zeros_h, out_hbm, aggS, sidx, didx, rows,
               gsem, ssem):
    cid = lax.axis_index("c")
    sid = lax.axis_index("s")

    @pl.when(sid == 0)
    def _():
        pltpu.sync_copy(zeros_h, aggS)

    pltpu.sync_copy(src2d.at[sid], sidx)
    pltpu.sync_copy(dst3d.at[sid], didx)
    off = cid * N

    @pl.loop(0, EPT // 16)
    def _(k):
        s = pl.ds(k * 16, 16)
        sidx[s] = sidx[s] + off

    plsc.subcore_barrier()

    def _gather(jj, b):
        return pltpu.make_async_copy(
            h2d.at[sidx.at[pl.ds(jj * C3, C3)]], rows.at[b], gsem.at[b]
        )

    def _scatter(jj, b):
        return pltpu.make_async_copy(
            rows.at[b], aggS.at[didx.at[jj]], ssem.at[b]
        )

    _gather(0, 0).start()

    @pl.loop(0, NCH3, step=2)
    def _(j):
        for sl in range(2):
            b2 = 1 - sl
            jj = j + sl

            @pl.when(jj < NCH3)
            def _():
                @pl.when(jj >= 1)
                def _():
                    _scatter(jj - 1, b2).wait()

                @pl.when(jj + 1 < NCH3)
                def _():
                    _gather(jj + 1, b2).start()

                _gather(jj, sl).wait()
                _scatter(jj, sl).start(add=True)

    _scatter(NCH3 - 1, (NCH3 - 1) % 2).wait()
    plsc.subcore_barrier()

    @pl.when(sid < N // RWB)
    def _():
        r0 = sid * RWB
        pltpu.sync_copy(aggS.at[pl.ds(r0, RWB)], out_hbm.at[pl.ds(cid * N + r0, RWB)])


def _final_body(a0, a1, degin, xr, nidx, cond, f2W, f2b, W2a, W2b, b2, g2, bt2, out):
    ss = jnp.dot(cond[...], f2W[...], preferred_element_type=jnp.float32) + f2b[...]
    gamma = ss[:, :D]
    beta = ss[:, D:]
    idx = nidx[0, 0, :]
    oh = (idx[:, None] == lax.broadcasted_iota(jnp.int32, (1, B), 1)).astype(jnp.float32)
    gg = jnp.dot(oh, gamma, preferred_element_type=jnp.float32)
    bb = jnp.dot(oh, beta, preferred_element_type=jnp.float32)
    norm = lax.rsqrt(jnp.maximum(degin[...], 1.0))
    h2 = (
        jnp.dot(a0[...], W2a[...], preferred_element_type=jnp.float32)
        + jnp.dot(a1[...], W2b[...], preferred_element_type=jnp.float32)
    ) * norm + b2[...]
    mu = jnp.mean(h2, axis=1, keepdims=True)
    xc = h2 - mu
    var = jnp.mean(xc * xc, axis=1, keepdims=True)
    ln = xc * lax.rsqrt(var + 1e-5) * g2[...] + bt2[...]
    f = ln * (1.0 + gg) + bb
    out[...] = f * jax.nn.sigmoid(f) + xr[...]


def _final(agg0, agg1, deg_in_tbl, x, n_index, cond, f2_W, f2_b, W2, b2, ln2_g, ln2_b):
    R = 1000
    rep = lambda i: (0, 0)
    return pl.pallas_call(
        _final_body,
        grid=(N // R,),
        in_specs=[
            pl.BlockSpec((R, H), lambda i: (i, 0)),
            pl.BlockSpec((R, H), lambda i: (i, 0)),
            pl.BlockSpec((R, 1), lambda i: (i, 0)),
            pl.BlockSpec((R, D), lambda i: (i, 0)),
            pl.BlockSpec((1, 1, R), lambda i: (i, 0, 0)),
            pl.BlockSpec((B, CD), rep),
            pl.BlockSpec((CD, 2 * D), rep),
            pl.BlockSpec((1, 2 * D), rep),
            pl.BlockSpec((H, D), rep),
            pl.BlockSpec((H, D), rep),
            pl.BlockSpec((1, D), rep),
            pl.BlockSpec((1, D), rep),
            pl.BlockSpec((1, D), rep),
        ],
        out_specs=pl.BlockSpec((R, D), lambda i: (i, 0)),
        out_shape=jax.ShapeDtypeStruct((N, D), jnp.float32),
    )(
        agg0, agg1, deg_in_tbl, x,
        n_index.reshape(N // R, 1, R),
        cond, f2_W, f2_b.reshape(1, 2 * D),
        W2[:H], W2[H:],
        b2.reshape(1, D), ln2_g.reshape(1, D), ln2_b.reshape(1, D),
    )


@functools.lru_cache(maxsize=None)
def _sc_calls():
    mesh = plsc.VectorSubcoreMesh(
        core_axis_name="c", subcore_axis_name="s", num_cores=2, num_subcores=NS
    )
    deg_call = pl.kernel(
        _deg_body,
        out_type=jax.ShapeDtypeStruct((2 * N,), jnp.float32),
        mesh=mesh,
        scratch_types=[
            pltpu.VMEM_SHARED((N,), jnp.float32),
            pltpu.VMEM((NCH1, C1), jnp.int32),
            pltpu.VMEM((C1,), jnp.float32),
            pltpu.VMEM((RWB,), jnp.float32),
        ],
    )
    scat_call = pl.kernel(
        _scat_body,
        out_type=jax.ShapeDtypeStruct((2 * N, H), jnp.float32),
        mesh=mesh,
        scratch_types=[
            pltpu.VMEM_SHARED((N, H), jnp.float32),
            pltpu.VMEM((EPT,), jnp.int32),
            pltpu.VMEM((NCH3, C3), jnp.int32),
            pltpu.VMEM((2, C3, H), jnp.float32),
            pltpu.SemaphoreType.DMA((2,)),
            pltpu.SemaphoreType.DMA((2,)),
        ],
    )
    return deg_call, scat_call


def kernel(edge_index, x, cond, n_index, W1, b1, ln1_g, ln1_b, f1_W, f1_b,
           W2, b2, ln2_g, ln2_b, f2_W, f2_b):
    _deg_call, _scat_call = _sc_calls()
    edge32 = edge_index.reshape(2 * NS, NCH1, C1)
    degtbl = _deg_call(
        edge32,
        jnp.zeros((N,), jnp.float32),
        jnp.ones((C1,), jnp.float32),
    )
    deg2d = degtbl.reshape(2 * N, 1)

    h2d = _prescale(x, deg2d[:N]).reshape(2 * N, H)

    src2d = edge_index[0].reshape(NS, EPT)
    dst3d = edge_index[1].reshape(NS, NCH3, C3)
    agg = _scat_call(src2d, dst3d, h2d, jnp.zeros((N, H), jnp.float32))

    return _final(agg[:N], agg[N:], deg2d[N:], x, n_index, cond,
                  f2_W, f2_b, W2, b2, ln2_g, ln2_b)


# speedup vs baseline: 8.6085x; 1.1600x over previous
import functools

import jax
import jax.numpy as jnp
from jax import lax
from jax.experimental import pallas as pl
from jax.experimental.pallas import tpu as pltpu
from jax.experimental.pallas import tpu_sc as plsc

N = 10000
E = 160000
D = 256
H = 128
CD = 128
B = 64

NS = 16
EPT = E // NS
C3 = 80
NCH3 = EPT // C3
C1 = 80
NCH1 = EPT // C1
RWB = 1000
NB = 10
R = N // NB


def _deg_body(edge1d, zeros1, ones_h, out_hbm, table, idx1d, ones_v, wb, sem):
    cid = lax.axis_index("c")
    sid = lax.axis_index("s")

    @pl.when(sid == 0)
    def _():
        pltpu.sync_copy(zeros1, table)

    pltpu.sync_copy(ones_h, ones_v)
    pltpu.sync_copy(edge1d.at[pl.ds(cid * E + sid * EPT, EPT)], idx1d)
    plsc.subcore_barrier()

    def _sc(j):
        return pltpu.make_async_copy(
            ones_v, table.at[idx1d.at[pl.ds(j * C1, C1)]], sem
        )

    @pl.loop(0, NCH1 - NCH1 % 8, step=8)
    def _(j):
        for b in range(8):
            _sc(j + b).start(add=True)
        for b in range(8):
            _sc(j + b).wait()

    for b in range(NCH1 % 8):
        _sc(NCH1 - NCH1 % 8 + b).start(add=True)
    for b in range(NCH1 % 8):
        _sc(NCH1 - NCH1 % 8 + b).wait()

    plsc.subcore_barrier()

    @pl.when(sid < N // RWB)
    def _():
        r0 = sid * RWB
        pltpu.sync_copy(table.at[pl.ds(r0, RWB)], wb)
        pltpu.sync_copy(wb, out_hbm.at[pl.ds(cid * N + r0, RWB)])


def _prescale_body(x_ref, deg_ref, out_ref):
    norm = lax.rsqrt(jnp.maximum(deg_ref[...], 1.0))
    out_ref[...] = x_ref[...] * norm


def _prescale(x, deg2d):
    return pl.pallas_call(
        _prescale_body,
        grid=(NB, 2),
        in_specs=[
            pl.BlockSpec((R, H), lambda i, c: (i, c)),
            pl.BlockSpec((R, 1), lambda i, c: (i, 0)),
        ],
        out_specs=pl.BlockSpec((R, H), lambda i, c: (c * NB + i, 0)),
        out_shape=jax.ShapeDtypeStruct((2 * N, H), jnp.float32),
    )(x, deg2d)


def _scat_body(edge1d, h2d, zeros_h, out_hbm, aggS, sidx, didx, rows, gsem, ssem):
    cid = lax.axis_index("c")
    sid = lax.axis_index("s")

    @pl.when(sid == 0)
    def _():
        pltpu.sync_copy(zeros_h, aggS)

    pltpu.sync_copy(edge1d.at[pl.ds(sid * EPT, EPT)], sidx)
    pltpu.sync_copy(edge1d.at[pl.ds(E + sid * EPT, EPT)], didx)
    off = cid * N

    @pl.loop(0, EPT // 16)
    def _(k):
        s = pl.ds(k * 16, 16)
        sidx[s] = sidx[s] + off

    plsc.subcore_barrier()

    def _gather(jj, b):
        return pltpu.make_async_copy(
            h2d.at[sidx.at[pl.ds(jj * C3, C3)]], rows.at[b], gsem.at[b]
        )

    def _scatter(jj, b):
        return pltpu.make_async_copy(
            rows.at[b], aggS.at[didx.at[pl.ds(jj * C3, C3)]], ssem.at[b]
        )

    _gather(0, 0).start()
    _gather(1, 1).start()

    @pl.loop(0, NCH3, step=3)
    def _(j):
        for sl in range(3):
            b2 = (sl + 2) % 3
            jj = j + sl

            @pl.when(jj < NCH3)
            def _():
                @pl.when(jj >= 1)
                def _():
                    _scatter(jj - 1, b2).wait()

                @pl.when(jj + 2 < NCH3)
                def _():
                    _gather(jj + 2, b2).start()

                _gather(jj, sl).wait()
                _scatter(jj, sl).start(add=True)

    _scatter(NCH3 - 1, (NCH3 - 1) % 3).wait()
    plsc.subcore_barrier()

    @pl.when(sid < N // RWB)
    def _():
        r0 = sid * RWB
        pltpu.sync_copy(aggS.at[pl.ds(r0, RWB)], out_hbm.at[pl.ds(cid * N + r0, RWB)])


def _final_body(a0, a1, degin, xr, nidx, cond, f2W, f2b, W2a, W2b, b2, g2, bt2, out):
    ss = jnp.dot(cond[...], f2W[...], preferred_element_type=jnp.float32) + f2b[...]
    gamma = ss[:, :D]
    beta = ss[:, D:]
    idx = nidx[0, 0, :]
    oh = (idx[:, None] == lax.broadcasted_iota(jnp.int32, (1, B), 1)).astype(jnp.float32)
    gg = jnp.dot(oh, gamma, preferred_element_type=jnp.float32)
    bb = jnp.dot(oh, beta, preferred_element_type=jnp.float32)
    norm = lax.rsqrt(jnp.maximum(degin[...], 1.0))
    h2 = (
        jnp.dot(a0[...], W2a[...], preferred_element_type=jnp.float32)
        + jnp.dot(a1[...], W2b[...], preferred_element_type=jnp.float32)
    ) * norm + b2[...]
    mu = jnp.mean(h2, axis=1, keepdims=True)
    xc = h2 - mu
    var = jnp.mean(xc * xc, axis=1, keepdims=True)
    ln = xc * lax.rsqrt(var + 1e-5) * g2[...] + bt2[...]
    f = ln * (1.0 + gg) + bb
    out[...] = f * jax.nn.sigmoid(f) + xr[...]


def _final(agg, deg2d, x, n_index, cond, f2_W, f2_b, W2, b2, ln2_g, ln2_b):
    rep = lambda i: (0, 0)
    return pl.pallas_call(
        _final_body,
        grid=(NB,),
        in_specs=[
            pl.BlockSpec((R, H), lambda i: (i, 0)),
            pl.BlockSpec((R, H), lambda i: (NB + i, 0)),
            pl.BlockSpec((R, 1), lambda i: (NB + i, 0)),
            pl.BlockSpec((R, D), lambda i: (i, 0)),
            pl.BlockSpec((1, 1, R), lambda i: (i, 0, 0)),
            pl.BlockSpec((B, CD), rep),
            pl.BlockSpec((CD, 2 * D), rep),
            pl.BlockSpec((1, 2 * D), rep),
            pl.BlockSpec((H, D), lambda i: (0, 0)),
            pl.BlockSpec((H, D), lambda i: (1, 0)),
            pl.BlockSpec((1, D), rep),
            pl.BlockSpec((1, D), rep),
            pl.BlockSpec((1, D), rep),
        ],
        out_specs=pl.BlockSpec((R, D), lambda i: (i, 0)),
        out_shape=jax.ShapeDtypeStruct((N, D), jnp.float32),
    )(
        agg, agg, deg2d, x,
        n_index.reshape(NB, 1, R),
        cond, f2_W, f2_b.reshape(1, 2 * D),
        W2, W2,
        b2.reshape(1, D), ln2_g.reshape(1, D), ln2_b.reshape(1, D),
    )


@functools.lru_cache(maxsize=None)
def _sc_calls():
    mesh = plsc.VectorSubcoreMesh(
        core_axis_name="c", subcore_axis_name="s", num_cores=2, num_subcores=NS
    )
    deg_call = pl.kernel(
        _deg_body,
        out_type=jax.ShapeDtypeStruct((2 * N,), jnp.float32),
        mesh=mesh,
        scratch_types=[
            pltpu.VMEM_SHARED((N,), jnp.float32),
            pltpu.VMEM((EPT,), jnp.int32),
            pltpu.VMEM((C1,), jnp.float32),
            pltpu.VMEM((RWB,), jnp.float32),
            pltpu.SemaphoreType.DMA,
        ],
    )
    scat_call = pl.kernel(
        _scat_body,
        out_type=jax.ShapeDtypeStruct((2 * N, H), jnp.float32),
        mesh=mesh,
        scratch_types=[
            pltpu.VMEM_SHARED((N, H), jnp.float32),
            pltpu.VMEM((EPT,), jnp.int32),
            pltpu.VMEM((EPT,), jnp.int32),
            pltpu.VMEM((3, C3, H), jnp.float32),
            pltpu.SemaphoreType.DMA((3,)),
            pltpu.SemaphoreType.DMA((3,)),
        ],
    )
    return deg_call, scat_call


def kernel(edge_index, x, cond, n_index, W1, b1, ln1_g, ln1_b, f1_W, f1_b,
           W2, b2, ln2_g, ln2_b, f2_W, f2_b):
    _deg_call, _scat_call = _sc_calls()
    edge1d = edge_index.reshape(2 * E)

    degtbl = _deg_call(
        edge1d,
        jnp.zeros((N,), jnp.float32),
        jnp.ones((C1,), jnp.float32),
    )
    deg2d = degtbl.reshape(2 * N, 1)

    h2d = _prescale(x, deg2d)

    agg = _scat_call(edge1d, h2d, jnp.zeros((N, H), jnp.float32))

    return _final(agg, deg2d, x, n_index, cond, f2_W, f2_b, W2, b2,
                  ln2_g, ln2_b)


# speedup vs baseline: 8.8043x; 1.0227x over previous
import functools

import jax
import jax.numpy as jnp
from jax import lax
from jax.experimental import pallas as pl
from jax.experimental.pallas import tpu as pltpu
from jax.experimental.pallas import tpu_sc as plsc

N = 10000
E = 160000
D = 256
H = 128
CD = 128
B = 64

NS = 16
EPT = E // NS
C3 = 80
NCH3 = EPT // C3
C1 = 80
NCH1 = EPT // C1
RWB = 1000
NB = 10
R = N // NB


def _deg_body(edge1d, zeros1, out_hbm, table, idx1d, ones_v, wb, sem):
    cid = lax.axis_index("c")
    sid = lax.axis_index("s")

    @pl.when(sid == 0)
    def _():
        pltpu.sync_copy(zeros1, table)

    for k in range(C1 // 16):
        ones_v[pl.ds(k * 16, 16)] = jnp.full((16,), 1.0, jnp.float32)
    pltpu.sync_copy(edge1d.at[pl.ds(cid * E + sid * EPT, EPT)], idx1d)
    plsc.subcore_barrier()

    def _sc(j):
        return pltpu.make_async_copy(
            ones_v, table.at[idx1d.at[pl.ds(j * C1, C1)]], sem
        )

    @pl.loop(0, NCH1 - NCH1 % 8, step=8)
    def _(j):
        for b in range(8):
            _sc(j + b).start(add=True)
        for b in range(8):
            _sc(j + b).wait()

    for b in range(NCH1 % 8):
        _sc(NCH1 - NCH1 % 8 + b).start(add=True)
    for b in range(NCH1 % 8):
        _sc(NCH1 - NCH1 % 8 + b).wait()

    plsc.subcore_barrier()

    @pl.when(sid < N // RWB)
    def _():
        r0 = sid * RWB
        pltpu.sync_copy(table.at[pl.ds(r0, RWB)], wb)
        pltpu.sync_copy(wb, out_hbm.at[pl.ds(cid * N + r0, RWB)])


def _prescale_body(x_ref, deg_ref, out_ref):
    norm = lax.rsqrt(jnp.maximum(deg_ref[...], 1.0))
    out_ref[...] = x_ref[...] * norm


def _prescale(x, deg2d):
    return pl.pallas_call(
        _prescale_body,
        grid=(NB, 2),
        in_specs=[
            pl.BlockSpec((R, H), lambda i, c: (i, c)),
            pl.BlockSpec((R, 1), lambda i, c: (i, 0)),
        ],
        out_specs=pl.BlockSpec((R, H), lambda i, c: (c * NB + i, 0)),
        out_shape=jax.ShapeDtypeStruct((2 * N, H), jnp.float32),
    )(x, deg2d)


def _scat_body(edge1d, h2d, zeros_h, out_hbm, aggS, sidx, didx, rows,
               gsem, ssem):
    cid = lax.axis_index("c")
    sid = lax.axis_index("s")

    @pl.when(sid < N // RWB)
    def _():
        r0 = sid * RWB
        pltpu.sync_copy(zeros_h.at[pl.ds(r0, RWB)], aggS.at[pl.ds(r0, RWB)])

    pltpu.sync_copy(edge1d.at[pl.ds(sid * EPT, EPT)], sidx)
    pltpu.sync_copy(edge1d.at[pl.ds(E + sid * EPT, EPT)], didx)
    off = cid * N

    def _fill_gidx(jj, b):
        for k in range(C3 // 16):
            sl_ = pl.ds(jj * C3 + k * 16, 16)
            sidx[sl_] = sidx[sl_] + off

    def _gather(jj, b):
        return pltpu.make_async_copy(
            h2d.at[sidx.at[pl.ds(jj * C3, C3)]], rows.at[b], gsem.at[b]
        )

    def _scatter(jj, b):
        return pltpu.make_async_copy(
            rows.at[b], aggS.at[didx.at[pl.ds(jj * C3, C3)]], ssem.at[b]
        )

    plsc.subcore_barrier()

    _fill_gidx(0, 0)
    _gather(0, 0).start()
    _fill_gidx(1, 1)
    _gather(1, 1).start()

    @pl.loop(0, NCH3, step=3)
    def _(j):
        for sl in range(3):
            b2 = (sl + 2) % 3
            jj = j + sl

            @pl.when(jj < NCH3)
            def _():
                @pl.when(jj >= 1)
                def _():
                    _scatter(jj - 1, b2).wait()

                @pl.when(jj + 2 < NCH3)
                def _():
                    _fill_gidx(jj + 2, b2)
                    _gather(jj + 2, b2).start()

                _gather(jj, sl).wait()
                _scatter(jj, sl).start(add=True)

    _scatter(NCH3 - 1, (NCH3 - 1) % 3).wait()
    plsc.subcore_barrier()

    @pl.when(sid < N // RWB)
    def _():
        r0 = sid * RWB
        pltpu.sync_copy(aggS.at[pl.ds(r0, RWB)], out_hbm.at[pl.ds(cid * N + r0, RWB)])


def _final_body(a0, a1, degin, xr, nidx, cond, f2W, f2b, W2a, W2b, b2, g2, bt2, out):
    ss = jnp.dot(cond[...], f2W[...], preferred_element_type=jnp.float32) + f2b[...]
    gamma = ss[:, :D]
    beta = ss[:, D:]
    idx = nidx[0, 0, :]
    oh = (idx[:, None] == lax.broadcasted_iota(jnp.int32, (1, B), 1)).astype(jnp.float32)
    gg = jnp.dot(oh, gamma, preferred_element_type=jnp.float32)
    bb = jnp.dot(oh, beta, preferred_element_type=jnp.float32)
    norm = lax.rsqrt(jnp.maximum(degin[...], 1.0))
    h2 = (
        jnp.dot(a0[...], W2a[...], preferred_element_type=jnp.float32)
        + jnp.dot(a1[...], W2b[...], preferred_element_type=jnp.float32)
    ) * norm + b2[...]
    mu = jnp.mean(h2, axis=1, keepdims=True)
    xc = h2 - mu
    var = jnp.mean(xc * xc, axis=1, keepdims=True)
    ln = xc * lax.rsqrt(var + 1e-5) * g2[...] + bt2[...]
    f = ln * (1.0 + gg) + bb
    out[...] = f * jax.nn.sigmoid(f) + xr[...]


def _final(agg, deg2d, x, n_index, cond, f2_W, f2_b, W2, b2, ln2_g, ln2_b):
    rep = lambda i: (0, 0)
    return pl.pallas_call(
        _final_body,
        grid=(NB,),
        in_specs=[
            pl.BlockSpec((R, H), lambda i: (i, 0)),
            pl.BlockSpec((R, H), lambda i: (NB + i, 0)),
            pl.BlockSpec((R, 1), lambda i: (NB + i, 0)),
            pl.BlockSpec((R, D), lambda i: (i, 0)),
            pl.BlockSpec((1, 1, R), lambda i: (i, 0, 0)),
            pl.BlockSpec((B, CD), rep),
            pl.BlockSpec((CD, 2 * D), rep),
            pl.BlockSpec((1, 2 * D), rep),
            pl.BlockSpec((H, D), lambda i: (0, 0)),
            pl.BlockSpec((H, D), lambda i: (1, 0)),
            pl.BlockSpec((1, D), rep),
            pl.BlockSpec((1, D), rep),
            pl.BlockSpec((1, D), rep),
        ],
        out_specs=pl.BlockSpec((R, D), lambda i: (i, 0)),
        out_shape=jax.ShapeDtypeStruct((N, D), jnp.float32),
    )(
        agg, agg, deg2d, x,
        n_index.reshape(NB, 1, R),
        cond, f2_W, f2_b.reshape(1, 2 * D),
        W2, W2,
        b2.reshape(1, D), ln2_g.reshape(1, D), ln2_b.reshape(1, D),
    )


@functools.lru_cache(maxsize=None)
def _sc_calls():
    mesh = plsc.VectorSubcoreMesh(
        core_axis_name="c", subcore_axis_name="s", num_cores=2, num_subcores=NS
    )
    deg_call = pl.kernel(
        _deg_body,
        out_type=jax.ShapeDtypeStruct((2 * N,), jnp.float32),
        mesh=mesh,
        scratch_types=[
            pltpu.VMEM_SHARED((N,), jnp.float32),
            pltpu.VMEM((EPT,), jnp.int32),
            pltpu.VMEM((C1,), jnp.float32),
            pltpu.VMEM((RWB,), jnp.float32),
            pltpu.SemaphoreType.DMA,
        ],
    )
    scat_call = pl.kernel(
        _scat_body,
        out_type=jax.ShapeDtypeStruct((2 * N, H), jnp.float32),
        mesh=mesh,
        scratch_types=[
            pltpu.VMEM_SHARED((N, H), jnp.float32),
            pltpu.VMEM((EPT,), jnp.int32),
            pltpu.VMEM((EPT,), jnp.int32),
            pltpu.VMEM((3, C3, H), jnp.float32),
            pltpu.SemaphoreType.DMA((3,)),
            pltpu.SemaphoreType.DMA((3,)),
        ],
    )
    return deg_call, scat_call


def kernel(edge_index, x, cond, n_index, W1, b1, ln1_g, ln1_b, f1_W, f1_b,
           W2, b2, ln2_g, ln2_b, f2_W, f2_b):
    _deg_call, _scat_call = _sc_calls()
    edge1d = edge_index.reshape(2 * E)

    degtbl = _deg_call(edge1d, jnp.zeros((N,), jnp.float32))
    deg2d = degtbl.reshape(2 * N, 1)

    h2d = _prescale(x, deg2d)

    agg = _scat_call(edge1d, h2d, jnp.zeros((N, H), jnp.float32))

    return _final(agg, deg2d, x, n_index, cond, f2_W, f2_b, W2, b2,
                  ln2_g, ln2_b)


# speedup vs baseline: 9.3962x; 1.0672x over previous
import functools

import jax
import jax.numpy as jnp
from jax import lax
from jax.experimental import pallas as pl
from jax.experimental.pallas import tpu as pltpu
from jax.experimental.pallas import tpu_sc as plsc

N = 10000
E = 160000
D = 256
H = 128
CD = 128
B = 64

NS = 16
EPT = E // NS
C3 = 80
NCH3 = EPT // C3
C1 = 80
NCH1 = EPT // C1
RWB = 1000
NB = 10
R = N // NB


def _deg_body(edge1d, zeros1, out_hbm, table, idx1d, ones_v, wb, sem):
    cid = lax.axis_index("c")
    sid = lax.axis_index("s")

    @pl.when(sid == 0)
    def _():
        pltpu.sync_copy(zeros1, table)

    for k in range(C1 // 16):
        ones_v[pl.ds(k * 16, 16)] = jnp.full((16,), 1.0, jnp.float32)
    pltpu.sync_copy(edge1d.at[pl.ds(cid * E + sid * EPT, EPT)], idx1d)
    plsc.subcore_barrier()

    def _sc(j):
        return pltpu.make_async_copy(
            ones_v, table.at[idx1d.at[pl.ds(j * C1, C1)]], sem
        )

    @pl.loop(0, NCH1 - NCH1 % 8, step=8)
    def _(j):
        for b in range(8):
            _sc(j + b).start(add=True)
        for b in range(8):
            _sc(j + b).wait()

    for b in range(NCH1 % 8):
        _sc(NCH1 - NCH1 % 8 + b).start(add=True)
    for b in range(NCH1 % 8):
        _sc(NCH1 - NCH1 % 8 + b).wait()

    plsc.subcore_barrier()

    @pl.when(sid < N // RWB)
    def _():
        r0 = sid * RWB
        pltpu.sync_copy(table.at[pl.ds(r0, RWB)], wb)
        pltpu.sync_copy(wb, out_hbm.at[pl.ds(cid * N + r0, RWB)])


def _prescale_body(x_ref, deg_ref, o0_ref, o1_ref):
    norm = lax.rsqrt(jnp.maximum(deg_ref[...], 1.0))
    h = x_ref[...] * norm
    o0_ref[...] = h[:, :H]
    o1_ref[...] = h[:, H:]


def _prescale(x, deg2d):
    PR = 2000
    return pl.pallas_call(
        _prescale_body,
        grid=(N // PR,),
        in_specs=[
            pl.BlockSpec((PR, D), lambda i: (i, 0)),
            pl.BlockSpec((PR, 1), lambda i: (i, 0)),
        ],
        out_specs=[
            pl.BlockSpec((PR, H), lambda i: (i, 0)),
            pl.BlockSpec((PR, H), lambda i: (i, 0)),
        ],
        out_shape=[
            jax.ShapeDtypeStruct((N, H), jnp.float32),
            jax.ShapeDtypeStruct((N, H), jnp.float32),
        ],
    )(x, deg2d)


def _scat_body(edge1d, h0, h1, zeros_s, out_hbm, aggS, sidx, didx, rows,
               gsem, ssem):
    cid = lax.axis_index("c")
    sid = lax.axis_index("s")

    @pl.when(sid < N // RWB)
    def _():
        r0 = sid * RWB
        pltpu.sync_copy(zeros_s, aggS.at[pl.ds(r0, RWB)])

    pltpu.sync_copy(edge1d.at[pl.ds(sid * EPT, EPT)], sidx)
    pltpu.sync_copy(edge1d.at[pl.ds(E + sid * EPT, EPT)], didx)

    def _gather_start(jj, b):
        isl = sidx.at[pl.ds(jj * C3, C3)]

        @pl.when(cid == 0)
        def _():
            pltpu.make_async_copy(h0.at[isl], rows.at[b], gsem.at[b]).start()

        @pl.when(cid == 1)
        def _():
            pltpu.make_async_copy(h1.at[isl], rows.at[b], gsem.at[b]).start()

    def _gather_wait(jj, b):
        pltpu.make_async_copy(
            h0.at[sidx.at[pl.ds(jj * C3, C3)]], rows.at[b], gsem.at[b]
        ).wait()

    def _scatter(jj, b):
        return pltpu.make_async_copy(
            rows.at[b], aggS.at[didx.at[pl.ds(jj * C3, C3)]], ssem.at[b]
        )

    plsc.subcore_barrier()

    _gather_start(0, 0)
    _gather_start(1, 1)

    @pl.loop(0, NCH3, step=3)
    def _(j):
        for sl in range(3):
            b2 = (sl + 2) % 3
            jj = j + sl

            @pl.when(jj < NCH3)
            def _():
                @pl.when(jj >= 1)
                def _():
                    _scatter(jj - 1, b2).wait()

                @pl.when(jj + 2 < NCH3)
                def _():
                    _gather_start(jj + 2, b2)

                _gather_wait(jj, sl)
                _scatter(jj, sl).start(add=True)

    _scatter(NCH3 - 1, (NCH3 - 1) % 3).wait()
    plsc.subcore_barrier()

    @pl.when(sid < N // RWB)
    def _():
        r0 = sid * RWB
        pltpu.sync_copy(aggS.at[pl.ds(r0, RWB)], out_hbm.at[pl.ds(cid * N + r0, RWB)])


def _final_body(a0, a1, degin, xr, nidx, cond, f2W, f2b, W2a, W2b, b2, g2, bt2, out):
    ss = jnp.dot(cond[...], f2W[...], preferred_element_type=jnp.float32) + f2b[...]
    gamma = ss[:, :D]
    beta = ss[:, D:]
    idx = nidx[0, 0, :]
    oh = (idx[:, None] == lax.broadcasted_iota(jnp.int32, (1, B), 1)).astype(jnp.float32)
    gg = jnp.dot(oh, gamma, preferred_element_type=jnp.float32)
    bb = jnp.dot(oh, beta, preferred_element_type=jnp.float32)
    norm = lax.rsqrt(jnp.maximum(degin[...], 1.0))
    h2 = (
        jnp.dot(a0[...], W2a[...], preferred_element_type=jnp.float32)
        + jnp.dot(a1[...], W2b[...], preferred_element_type=jnp.float32)
    ) * norm + b2[...]
    mu = jnp.mean(h2, axis=1, keepdims=True)
    xc = h2 - mu
    var = jnp.mean(xc * xc, axis=1, keepdims=True)
    ln = xc * lax.rsqrt(var + 1e-5) * g2[...] + bt2[...]
    f = ln * (1.0 + gg) + bb
    out[...] = f * jax.nn.sigmoid(f) + xr[...]


def _final(agg, deg2d, x, n_index, cond, f2_W, f2_b, W2, b2, ln2_g, ln2_b):
    rep = lambda i: (0, 0)
    return pl.pallas_call(
        _final_body,
        grid=(NB,),
        in_specs=[
            pl.BlockSpec((R, H), lambda i: (i, 0)),
            pl.BlockSpec((R, H), lambda i: (NB + i, 0)),
            pl.BlockSpec((R, 1), lambda i: (NB + i, 0)),
            pl.BlockSpec((R, D), lambda i: (i, 0)),
            pl.BlockSpec((1, 1, R), lambda i: (i, 0, 0)),
            pl.BlockSpec((B, CD), rep),
            pl.BlockSpec((CD, 2 * D), rep),
            pl.BlockSpec((1, 2 * D), rep),
            pl.BlockSpec((H, D), lambda i: (0, 0)),
            pl.BlockSpec((H, D), lambda i: (1, 0)),
            pl.BlockSpec((1, D), rep),
            pl.BlockSpec((1, D), rep),
            pl.BlockSpec((1, D), rep),
        ],
        out_specs=pl.BlockSpec((R, D), lambda i: (i, 0)),
        out_shape=jax.ShapeDtypeStruct((N, D), jnp.float32),
    )(
        agg, agg, deg2d, x,
        n_index.reshape(NB, 1, R),
        cond, f2_W, f2_b.reshape(1, 2 * D),
        W2, W2,
        b2.reshape(1, D), ln2_g.reshape(1, D), ln2_b.reshape(1, D),
    )


@functools.lru_cache(maxsize=None)
def _sc_calls():
    mesh = plsc.VectorSubcoreMesh(
        core_axis_name="c", subcore_axis_name="s", num_cores=2, num_subcores=NS
    )
    deg_call = pl.kernel(
        _deg_body,
        out_type=jax.ShapeDtypeStruct((2 * N,), jnp.float32),
        mesh=mesh,
        scratch_types=[
            pltpu.VMEM_SHARED((N,), jnp.float32),
            pltpu.VMEM((EPT,), jnp.int32),
            pltpu.VMEM((C1,), jnp.float32),
            pltpu.VMEM((RWB,), jnp.float32),
            pltpu.SemaphoreType.DMA,
        ],
    )
    scat_call = pl.kernel(
        _scat_body,
        out_type=jax.ShapeDtypeStruct((2 * N, H), jnp.float32),
        mesh=mesh,
        scratch_types=[
            pltpu.VMEM_SHARED((N, H), jnp.float32),
            pltpu.VMEM((EPT,), jnp.int32),
            pltpu.VMEM((EPT,), jnp.int32),
            pltpu.VMEM((3, C3, H), jnp.float32),
            pltpu.SemaphoreType.DMA((3,)),
            pltpu.SemaphoreType.DMA((3,)),
        ],
    )
    return deg_call, scat_call


def kernel(edge_index, x, cond, n_index, W1, b1, ln1_g, ln1_b, f1_W, f1_b,
           W2, b2, ln2_g, ln2_b, f2_W, f2_b):
    _deg_call, _scat_call = _sc_calls()
    edge1d = edge_index.reshape(2 * E)

    degtbl = _deg_call(edge1d, jnp.zeros((N,), jnp.float32))
    deg2d = degtbl.reshape(2 * N, 1)

    h0, h1 = _prescale(x, deg2d)

    agg = _scat_call(edge1d, h0, h1, jnp.zeros((RWB, H), jnp.float32))

    return _final(agg, deg2d, x, n_index, cond, f2_W, f2_b, W2, b2,
                  ln2_g, ln2_b)


# speedup vs baseline: 9.4797x; 1.0089x over previous
import functools

import jax
import jax.numpy as jnp
from jax import lax
from jax.experimental import pallas as pl
from jax.experimental.pallas import tpu as pltpu
from jax.experimental.pallas import tpu_sc as plsc

N = 10000
E = 160000
D = 256
H = 128
CD = 128
B = 64

NS = 16
EPT = E // NS
C3 = 80
NCH3 = EPT // C3
C1 = 80
NCH1 = EPT // C1
RWB = 1000
NB = 10
R = N // NB


def _deg_body(edge1d, zeros1, out_hbm, table, idx1d, ones_v, wb, sem):
    cid = lax.axis_index("c")
    sid = lax.axis_index("s")

    @pl.when(sid == 0)
    def _():
        pltpu.sync_copy(zeros1, table)

    for k in range(C1 // 16):
        ones_v[pl.ds(k * 16, 16)] = jnp.full((16,), 1.0, jnp.float32)
    pltpu.sync_copy(edge1d.at[pl.ds(cid * E + sid * EPT, EPT)], idx1d)
    plsc.subcore_barrier()

    def _sc(j):
        return pltpu.make_async_copy(
            ones_v, table.at[idx1d.at[pl.ds(j * C1, C1)]], sem
        )

    @pl.loop(0, NCH1 - NCH1 % 8, step=8)
    def _(j):
        for b in range(8):
            _sc(j + b).start(add=True)
        for b in range(8):
            _sc(j + b).wait()

    for b in range(NCH1 % 8):
        _sc(NCH1 - NCH1 % 8 + b).start(add=True)
    for b in range(NCH1 % 8):
        _sc(NCH1 - NCH1 % 8 + b).wait()

    plsc.subcore_barrier()

    @pl.when(sid < N // RWB)
    def _():
        r0 = sid * RWB
        pltpu.sync_copy(table.at[pl.ds(r0, RWB)], wb)
        pltpu.sync_copy(wb, out_hbm.at[pl.ds(cid * N + r0, RWB)])


def _prescale_body(x_ref, deg_ref, o0_ref, o1_ref):
    norm = lax.rsqrt(jnp.maximum(deg_ref[...], 1.0))
    h = x_ref[...] * norm
    o0_ref[...] = h[:, :H]
    o1_ref[...] = h[:, H:]


def _prescale(x, deg2d):
    PR = 2000
    return pl.pallas_call(
        _prescale_body,
        grid=(N // PR,),
        in_specs=[
            pl.BlockSpec((PR, D), lambda i: (i, 0)),
            pl.BlockSpec((PR, 1), lambda i: (i, 0)),
        ],
        out_specs=[
            pl.BlockSpec((PR, H), lambda i: (i, 0)),
            pl.BlockSpec((PR, H), lambda i: (i, 0)),
        ],
        out_shape=[
            jax.ShapeDtypeStruct((N, H), jnp.float32),
            jax.ShapeDtypeStruct((N, H), jnp.float32),
        ],
    )(x, deg2d)


def _scat_body(edge1d, h0, h1, zeros_s, out_hbm, aggS, sidx, didx, rows,
               gsem, ssem):
    cid = lax.axis_index("c")
    sid = lax.axis_index("s")

    @pl.when(sid < N // RWB)
    def _():
        r0 = sid * RWB
        pltpu.sync_copy(zeros_s, aggS.at[pl.ds(r0, RWB)])

    pltpu.sync_copy(edge1d.at[pl.ds(sid * EPT, EPT)], sidx)
    pltpu.sync_copy(edge1d.at[pl.ds(E + sid * EPT, EPT)], didx)

    def _gather_start(jj, b):
        isl = sidx.at[pl.ds(jj * C3, C3)]

        @pl.when(cid == 0)
        def _():
            pltpu.make_async_copy(h0.at[isl], rows.at[b], gsem.at[b]).start()

        @pl.when(cid == 1)
        def _():
            pltpu.make_async_copy(h1.at[isl], rows.at[b], gsem.at[b]).start()

    def _gather_wait(jj, b):
        pltpu.make_async_copy(
            h0.at[sidx.at[pl.ds(jj * C3, C3)]], rows.at[b], gsem.at[b]
        ).wait()

    def _scatter(jj, b):
        return pltpu.make_async_copy(
            rows.at[b], aggS.at[didx.at[pl.ds(jj * C3, C3)]], ssem.at[b]
        )

    plsc.subcore_barrier()

    _gather_start(0, 0)
    _gather_start(1, 1)

    @pl.loop(0, NCH3, step=3)
    def _(j):
        for sl in range(3):
            b2 = (sl + 2) % 3
            jj = j + sl

            @pl.when(jj < NCH3)
            def _():
                @pl.when(jj >= 1)
                def _():
                    _scatter(jj - 1, b2).wait()

                @pl.when(jj + 2 < NCH3)
                def _():
                    _gather_start(jj + 2, b2)

                _gather_wait(jj, sl)
                _scatter(jj, sl).start(add=True)

    _scatter(NCH3 - 1, (NCH3 - 1) % 3).wait()
    plsc.subcore_barrier()

    @pl.when(sid < N // RWB)
    def _():
        r0 = sid * RWB
        pltpu.sync_copy(aggS.at[pl.ds(r0, RWB)], out_hbm.at[pl.ds(cid * N + r0, RWB)])


def _final_body(a0, a1, degin, xr, nidx, cond, f2W, f2b, W2a, W2b, b2, g2, bt2, out):
    ss = jnp.dot(cond[...], f2W[...], preferred_element_type=jnp.float32) + f2b[...]
    gamma = ss[:, :D]
    beta = ss[:, D:]
    idx = nidx[0, 0, :]
    oh = (idx[:, None] == lax.broadcasted_iota(jnp.int32, (1, B), 1)).astype(jnp.float32)
    gg = jnp.dot(oh, gamma, preferred_element_type=jnp.float32)
    bb = jnp.dot(oh, beta, preferred_element_type=jnp.float32)
    norm = lax.rsqrt(jnp.maximum(degin[...], 1.0))
    h2 = (
        jnp.dot(a0[...], W2a[...], preferred_element_type=jnp.float32)
        + jnp.dot(a1[...], W2b[...], preferred_element_type=jnp.float32)
    ) * norm + b2[...]
    mu = jnp.mean(h2, axis=1, keepdims=True)
    xc = h2 - mu
    var = jnp.mean(xc * xc, axis=1, keepdims=True)
    ln = xc * lax.rsqrt(var + 1e-5) * g2[...] + bt2[...]
    f = ln * (1.0 + gg) + bb
    out[...] = f * jax.nn.sigmoid(f) + xr[...]


def _final(agg, deg2d, x, n_index, cond, f2_W, f2_b, W2, b2, ln2_g, ln2_b):
    FR = 2000
    FNB = N // FR
    rep = lambda i: (0, 0)
    return pl.pallas_call(
        _final_body,
        grid=(FNB,),
        in_specs=[
            pl.BlockSpec((FR, H), lambda i: (i, 0)),
            pl.BlockSpec((FR, H), lambda i: (FNB + i, 0)),
            pl.BlockSpec((FR, 1), lambda i: (FNB + i, 0)),
            pl.BlockSpec((FR, D), lambda i: (i, 0)),
            pl.BlockSpec((1, 1, FR), lambda i: (i, 0, 0)),
            pl.BlockSpec((B, CD), rep),
            pl.BlockSpec((CD, 2 * D), rep),
            pl.BlockSpec((1, 2 * D), rep),
            pl.BlockSpec((H, D), lambda i: (0, 0)),
            pl.BlockSpec((H, D), lambda i: (1, 0)),
            pl.BlockSpec((1, D), rep),
            pl.BlockSpec((1, D), rep),
            pl.BlockSpec((1, D), rep),
        ],
        out_specs=pl.BlockSpec((FR, D), lambda i: (i, 0)),
        out_shape=jax.ShapeDtypeStruct((N, D), jnp.float32),
    )(
        agg, agg, deg2d, x,
        n_index.reshape(FNB, 1, FR),
        cond, f2_W, f2_b.reshape(1, 2 * D),
        W2, W2,
        b2.reshape(1, D), ln2_g.reshape(1, D), ln2_b.reshape(1, D),
    )


@functools.lru_cache(maxsize=None)
def _sc_calls():
    mesh = plsc.VectorSubcoreMesh(
        core_axis_name="c", subcore_axis_name="s", num_cores=2, num_subcores=NS
    )
    deg_call = pl.kernel(
        _deg_body,
        out_type=jax.ShapeDtypeStruct((2 * N,), jnp.float32),
        mesh=mesh,
        scratch_types=[
            pltpu.VMEM_SHARED((N,), jnp.float32),
            pltpu.VMEM((EPT,), jnp.int32),
            pltpu.VMEM((C1,), jnp.float32),
            pltpu.VMEM((RWB,), jnp.float32),
            pltpu.SemaphoreType.DMA,
        ],
    )
    scat_call = pl.kernel(
        _scat_body,
        out_type=jax.ShapeDtypeStruct((2 * N, H), jnp.float32),
        mesh=mesh,
        scratch_types=[
            pltpu.VMEM_SHARED((N, H), jnp.float32),
            pltpu.VMEM((EPT,), jnp.int32),
            pltpu.VMEM((EPT,), jnp.int32),
            pltpu.VMEM((3, C3, H), jnp.float32),
            pltpu.SemaphoreType.DMA((3,)),
            pltpu.SemaphoreType.DMA((3,)),
        ],
    )
    return deg_call, scat_call


def kernel(edge_index, x, cond, n_index, W1, b1, ln1_g, ln1_b, f1_W, f1_b,
           W2, b2, ln2_g, ln2_b, f2_W, f2_b):
    _deg_call, _scat_call = _sc_calls()
    edge1d = edge_index.reshape(2 * E)

    degtbl = _deg_call(edge1d, jnp.zeros((N,), jnp.float32))
    deg2d = degtbl.reshape(2 * N, 1)

    h0, h1 = _prescale(x, deg2d)

    agg = _scat_call(edge1d, h0, h1, jnp.zeros((RWB, H), jnp.float32))

    return _final(agg, deg2d, x, n_index, cond, f2_W, f2_b, W2, b2,
                  ln2_g, ln2_b)


# speedup vs baseline: 10.0758x; 1.0629x over previous
import functools

import jax
import jax.numpy as jnp
from jax import lax
from jax.experimental import pallas as pl
from jax.experimental.pallas import tpu as pltpu
from jax.experimental.pallas import tpu_sc as plsc

N = 10000
E = 160000
D = 256
H = 128
CD = 128
B = 64

NS = 16
EPT = E // NS
C3 = 80
NCH3 = EPT // C3
C1 = 80
NCH1 = EPT // C1
RWB = 1000
NB = 10
R = N // NB


def _deg_body(edge1d, zeros1, out_hbm, table, idx1d, ones_v, wb, sem):
    cid = lax.axis_index("c")
    sid = lax.axis_index("s")

    @pl.when(sid == 0)
    def _():
        pltpu.sync_copy(zeros1, table)

    for k in range(C1 // 16):
        ones_v[pl.ds(k * 16, 16)] = jnp.full((16,), 1.0, jnp.float32)
    pltpu.sync_copy(edge1d.at[pl.ds(cid * E + sid * EPT, EPT)], idx1d)
    plsc.subcore_barrier()

    def _sc(j):
        return pltpu.make_async_copy(
            ones_v, table.at[idx1d.at[pl.ds(j * C1, C1)]], sem
        )

    @pl.loop(0, NCH1 - NCH1 % 8, step=8)
    def _(j):
        for b in range(8):
            _sc(j + b).start(add=True)
        for b in range(8):
            _sc(j + b).wait()

    for b in range(NCH1 % 8):
        _sc(NCH1 - NCH1 % 8 + b).start(add=True)
    for b in range(NCH1 % 8):
        _sc(NCH1 - NCH1 % 8 + b).wait()

    plsc.subcore_barrier()

    @pl.when(sid < N // RWB)
    def _():
        r0 = sid * RWB
        pltpu.sync_copy(table.at[pl.ds(r0, RWB)], wb)
        pltpu.sync_copy(wb, out_hbm.at[pl.ds(cid * N + r0, RWB)])


def _prescale_body(x_ref, deg_ref, o0_ref, o1_ref):
    deg = deg_ref[0, 0, :]
    norm = lax.rsqrt(jnp.maximum(deg, 1.0))[:, None]
    h = x_ref[...] * norm
    o0_ref[...] = h[:, :H]
    o1_ref[...] = h[:, H:]


def _prescale(x, deg3d):
    PR = 2000
    return pl.pallas_call(
        _prescale_body,
        grid=(N // PR,),
        in_specs=[
            pl.BlockSpec((PR, D), lambda i: (i, 0)),
            pl.BlockSpec((1, 1, PR), lambda i: (i, 0, 0)),
        ],
        out_specs=[
            pl.BlockSpec((PR, H), lambda i: (i, 0)),
            pl.BlockSpec((PR, H), lambda i: (i, 0)),
        ],
        out_shape=[
            jax.ShapeDtypeStruct((N, H), jnp.float32),
            jax.ShapeDtypeStruct((N, H), jnp.float32),
        ],
    )(x, deg3d)


def _scat_body(edge1d, h0, h1, zeros_s, out_hbm, aggS, sidx, didx, rows,
               gsem, ssem):
    cid = lax.axis_index("c")
    sid = lax.axis_index("s")

    @pl.when(sid < N // RWB)
    def _():
        r0 = sid * RWB
        pltpu.sync_copy(zeros_s, aggS.at[pl.ds(r0, RWB)])

    pltpu.sync_copy(edge1d.at[pl.ds(sid * EPT, EPT)], sidx)
    pltpu.sync_copy(edge1d.at[pl.ds(E + sid * EPT, EPT)], didx)

    def _gather_start(jj, b):
        isl = sidx.at[pl.ds(jj * C3, C3)]

        @pl.when(cid == 0)
        def _():
            pltpu.make_async_copy(h0.at[isl], rows.at[b], gsem.at[b]).start()

        @pl.when(cid == 1)
        def _():
            pltpu.make_async_copy(h1.at[isl], rows.at[b], gsem.at[b]).start()

    def _gather_wait(jj, b):
        pltpu.make_async_copy(
            h0.at[sidx.at[pl.ds(jj * C3, C3)]], rows.at[b], gsem.at[b]
        ).wait()

    def _scatter(jj, b):
        return pltpu.make_async_copy(
            rows.at[b], aggS.at[didx.at[pl.ds(jj * C3, C3)]], ssem.at[b]
        )

    plsc.subcore_barrier()

    _gather_start(0, 0)
    _gather_start(1, 1)

    @pl.loop(0, NCH3, step=3)
    def _(j):
        for sl in range(3):
            b2 = (sl + 2) % 3
            jj = j + sl

            @pl.when(jj < NCH3)
            def _():
                @pl.when(jj >= 1)
                def _():
                    _scatter(jj - 1, b2).wait()

                @pl.when(jj + 2 < NCH3)
                def _():
                    _gather_start(jj + 2, b2)

                _gather_wait(jj, sl)
                _scatter(jj, sl).start(add=True)

    _scatter(NCH3 - 1, (NCH3 - 1) % 3).wait()
    plsc.subcore_barrier()

    @pl.when(sid < N // RWB)
    def _():
        r0 = sid * RWB
        pltpu.sync_copy(aggS.at[pl.ds(r0, RWB)], out_hbm.at[pl.ds(cid * N + r0, RWB)])


def _final_body(a0, a1, degin, xr, nidx, cond, f2W, f2b, W2a, W2b, b2, g2, bt2, out):
    ss = jnp.dot(cond[...], f2W[...], preferred_element_type=jnp.float32) + f2b[...]
    gamma = ss[:, :D]
    beta = ss[:, D:]
    idx = nidx[0, 0, :]
    oh = (idx[:, None] == lax.broadcasted_iota(jnp.int32, (1, B), 1)).astype(jnp.float32)
    gg = jnp.dot(oh, gamma, preferred_element_type=jnp.float32)
    bb = jnp.dot(oh, beta, preferred_element_type=jnp.float32)
    norm = lax.rsqrt(jnp.maximum(degin[0, 0, :], 1.0))[:, None]
    h2 = (
        jnp.dot(a0[...], W2a[...], preferred_element_type=jnp.float32)
        + jnp.dot(a1[...], W2b[...], preferred_element_type=jnp.float32)
    ) * norm + b2[...]
    mu = jnp.mean(h2, axis=1, keepdims=True)
    xc = h2 - mu
    var = jnp.mean(xc * xc, axis=1, keepdims=True)
    ln = xc * lax.rsqrt(var + 1e-5) * g2[...] + bt2[...]
    f = ln * (1.0 + gg) + bb
    out[...] = f * jax.nn.sigmoid(f) + xr[...]


def _final(agg, deg3d, x, n_index, cond, f2_W, f2_b, W2, b2, ln2_g, ln2_b):
    FR = 2000
    FNB = N // FR
    rep = lambda i: (0, 0)
    return pl.pallas_call(
        _final_body,
        grid=(FNB,),
        in_specs=[
            pl.BlockSpec((FR, H), lambda i: (i, 0)),
            pl.BlockSpec((FR, H), lambda i: (FNB + i, 0)),
            pl.BlockSpec((1, 1, FR), lambda i: (FNB + i, 0, 0)),
            pl.BlockSpec((FR, D), lambda i: (i, 0)),
            pl.BlockSpec((1, 1, FR), lambda i: (i, 0, 0)),
            pl.BlockSpec((B, CD), rep),
            pl.BlockSpec((CD, 2 * D), rep),
            pl.BlockSpec((1, 2 * D), rep),
            pl.BlockSpec((H, D), lambda i: (0, 0)),
            pl.BlockSpec((H, D), lambda i: (1, 0)),
            pl.BlockSpec((1, D), rep),
            pl.BlockSpec((1, D), rep),
            pl.BlockSpec((1, D), rep),
        ],
        out_specs=pl.BlockSpec((FR, D), lambda i: (i, 0)),
        out_shape=jax.ShapeDtypeStruct((N, D), jnp.float32),
    )(
        agg, agg, deg3d, x,
        n_index.reshape(FNB, 1, FR),
        cond, f2_W, f2_b.reshape(1, 2 * D),
        W2, W2,
        b2.reshape(1, D), ln2_g.reshape(1, D), ln2_b.reshape(1, D),
    )


@functools.lru_cache(maxsize=None)
def _sc_calls():
    mesh = plsc.VectorSubcoreMesh(
        core_axis_name="c", subcore_axis_name="s", num_cores=2, num_subcores=NS
    )
    deg_call = pl.kernel(
        _deg_body,
        out_type=jax.ShapeDtypeStruct((2 * N,), jnp.float32),
        mesh=mesh,
        scratch_types=[
            pltpu.VMEM_SHARED((N,), jnp.float32),
            pltpu.VMEM((EPT,), jnp.int32),
            pltpu.VMEM((C1,), jnp.float32),
            pltpu.VMEM((RWB,), jnp.float32),
            pltpu.SemaphoreType.DMA,
        ],
    )
    scat_call = pl.kernel(
        _scat_body,
        out_type=jax.ShapeDtypeStruct((2 * N, H), jnp.float32),
        mesh=mesh,
        scratch_types=[
            pltpu.VMEM_SHARED((N, H), jnp.float32),
            pltpu.VMEM((EPT,), jnp.int32),
            pltpu.VMEM((EPT,), jnp.int32),
            pltpu.VMEM((3, C3, H), jnp.float32),
            pltpu.SemaphoreType.DMA((3,)),
            pltpu.SemaphoreType.DMA((3,)),
        ],
    )
    return deg_call, scat_call


def kernel(edge_index, x, cond, n_index, W1, b1, ln1_g, ln1_b, f1_W, f1_b,
           W2, b2, ln2_g, ln2_b, f2_W, f2_b):
    _deg_call, _scat_call = _sc_calls()
    edge1d = edge_index.reshape(2 * E)

    degtbl = _deg_call(edge1d, jnp.zeros((N,), jnp.float32))
    deg3d = degtbl.reshape(2 * (N // 2000), 1, 2000)

    h0, h1 = _prescale(x, deg3d)

    agg = _scat_call(edge1d, h0, h1, jnp.zeros((RWB, H), jnp.float32))

    return _final(agg, deg3d, x, n_index, cond, f2_W, f2_b, W2, b2,
                  ln2_g, ln2_b)
